# baseline (speedup 1.0000x reference)
import jax
import jax.numpy as jnp
from jax import lax
from jax.experimental import pallas as pl
from jax.experimental.pallas import tpu as pltpu
from jax.experimental.pallas import tpu_sc as plsc

N_NODES = 10000
D_IN = 128
D_OUT = 32
D_CAT = 2 * D_OUT
N_EDGES = 320000

NC = 2
NS = 16
NW = NC * NS

EPT = N_EDGES // NW
BB = 128
NBF = EPT // BB
TAIL = EPT - NBF * BB
NBUF = 3

ACC_ROWS = N_NODES + 16
ROWS_PER_TILE = N_NODES // NS
ZCHUNK = 125

DEG_ROWS = 10240
DEG_PER_TILE = DEG_ROWS // NS

_mesh = plsc.VectorSubcoreMesh(
    core_axis_name="c", subcore_axis_name="s", num_cores=NC, num_subcores=NS
)

_sc_params = pltpu.CompilerParams(use_tc_tiling_on_sc=False)


NDSEM = 6


def _deg_body(edge_hbm, z_hbm, ones_hbm, degp_hbm, idx_v, zb_v, ones_v, *rest):
    dsem = rest[:NDSEM]
    deg_sp = rest[NDSEM]
    c = lax.axis_index("c")
    s = lax.axis_index("s")
    w = c * NS + s
    pltpu.sync_copy(edge_hbm.at[1, pl.ds(w * EPT, EPT)], idx_v)
    pltpu.sync_copy(z_hbm, zb_v)
    pltpu.sync_copy(ones_hbm, ones_v)
    pltpu.sync_copy(zb_v.at[pl.ds(0, DEG_PER_TILE)],
                    deg_sp.at[pl.ds(s * DEG_PER_TILE, DEG_PER_TILE)])
    plsc.subcore_barrier()

    def flight(u, carry):
        descs = []
        for b in range(NDSEM):
            j = u * NDSEM + b
            descs.append(pltpu.async_copy(
                ones_v, deg_sp.at[idx_v.at[pl.ds(j * BB, BB)]], dsem[b], add=True))
        for d in descs:
            d.wait()
        return carry

    lax.fori_loop(0, NBF // NDSEM, flight, 0)
    pltpu.sync_copy(ones_v.at[pl.ds(0, TAIL)],
                    deg_sp.at[idx_v.at[pl.ds(NBF * BB, TAIL)]], add=True)
    plsc.subcore_barrier()
    pltpu.sync_copy(deg_sp.at[pl.ds(s * DEG_PER_TILE, DEG_PER_TILE)], zb_v)
    pltpu.sync_copy(zb_v, degp_hbm.at[c, pl.ds(s * DEG_PER_TILE, DEG_PER_TILE)])


_deg_kernel = pl.kernel(
    _deg_body,
    out_type=jax.ShapeDtypeStruct((NC, DEG_ROWS), jnp.float32),
    mesh=_mesh,
    scratch_types=[
        pltpu.VMEM((EPT,), jnp.int32),
        pltpu.VMEM((DEG_PER_TILE,), jnp.float32),
        pltpu.VMEM((BB,), jnp.float32),
        *[pltpu.SemaphoreType.DMA for _ in range(NDSEM)],
        pltpu.VMEM_SHARED((DEG_ROWS,), jnp.float32),
    ],
    compiler_params=_sc_params,
)


def _pre_body(x_ref, w_ref, degp_ref, g_ref):
    dp = degp_ref[...]
    deg = dp[:, 0] + dp[:, 1] + 1.0
    dinv = lax.rsqrt(deg)
    h = jnp.dot(x_ref[...], w_ref[...], preferred_element_type=jnp.float32)
    g_ref[...] = h * dinv[:, None]


def _pre(x, w_cat, degp):
    blk = 2000
    return pl.pallas_call(
        _pre_body,
        grid=(N_NODES // blk,),
        in_specs=[
            pl.BlockSpec((blk, D_IN), lambda i: (i, 0)),
            pl.BlockSpec((D_IN, D_CAT), lambda i: (0, 0)),
            pl.BlockSpec((blk, NC), lambda i: (i, 0)),
        ],
        out_specs=pl.BlockSpec((blk, D_CAT), lambda i: (i, 0)),
        out_shape=jax.ShapeDtypeStruct((N_NODES, D_CAT), jnp.float32),
    )(x, w_cat, degp)


def _segsum_body(edge_hbm, g_hbm, z_hbm, accp_hbm, sidx_v, didx_v, *rest):
    rows = rest[:NBUF]
    gsem = rest[NBUF:2 * NBUF]
    ssem = rest[2 * NBUF:3 * NBUF]
    acc_sp = rest[3 * NBUF]
    g_sp = rest[3 * NBUF + 1]
    c = lax.axis_index("c")
    s = lax.axis_index("s")
    w = c * NS + s
    pltpu.sync_copy(edge_hbm.at[0, pl.ds(w * EPT, EPT)], sidx_v)
    pltpu.sync_copy(edge_hbm.at[1, pl.ds(w * EPT, EPT)], didx_v)
    base = s * ROWS_PER_TILE
    pltpu.sync_copy(g_hbm.at[pl.ds(base, ROWS_PER_TILE)],
                    g_sp.at[pl.ds(base, ROWS_PER_TILE)])
    pltpu.sync_copy(z_hbm, rows[0].at[pl.ds(0, ZCHUNK)])
    for r in range(ROWS_PER_TILE // ZCHUNK):
        pltpu.sync_copy(rows[0].at[pl.ds(0, ZCHUNK)],
                        acc_sp.at[pl.ds(base + r * ZCHUNK, ZCHUNK)])
    plsc.subcore_barrier()
    for b in range(NBUF):
        pltpu.async_copy(g_sp.at[sidx_v.at[pl.ds(b * BB, BB)]], rows[b], gsem[b])

    def group(u, carry):
        sdescs = []
        for b in range(NBUF):
            j = u * NBUF + b
            pltpu.make_async_copy(
                g_sp.at[sidx_v.at[pl.ds(j * BB, BB)]], rows[b], gsem[b]).wait()
            sdescs.append(pltpu.async_copy(
                rows[b], acc_sp.at[didx_v.at[pl.ds(j * BB, BB)]], ssem[b], add=True))
        for b in range(NBUF):
            j2 = (u + 1) * NBUF + b
            sdescs[b].wait()
            pltpu.async_copy(g_sp.at[sidx_v.at[pl.ds(j2 * BB, BB)]], rows[b], gsem[b])
        return carry

    lax.fori_loop(0, NBF // NBUF - 1, group, 0)
    tdescs = []
    for b in range(NBUF):
        j = NBF - NBUF + b
        pltpu.make_async_copy(
            g_sp.at[sidx_v.at[pl.ds(j * BB, BB)]], rows[b], gsem[b]).wait()
        tdescs.append(pltpu.async_copy(
            rows[b], acc_sp.at[didx_v.at[pl.ds(j * BB, BB)]], ssem[b], add=True))
    for d in tdescs:
        d.wait()
    pltpu.sync_copy(g_sp.at[sidx_v.at[pl.ds(NBF * BB, TAIL)]],
                    rows[0].at[pl.ds(0, TAIL)])
    pltpu.sync_copy(rows[0].at[pl.ds(0, TAIL)],
                    acc_sp.at[didx_v.at[pl.ds(NBF * BB, TAIL)]], add=True)
    plsc.subcore_barrier()
    for r in range(ROWS_PER_TILE // ZCHUNK):
        b = r % NBUF
        pltpu.sync_copy(acc_sp.at[pl.ds(base + r * ZCHUNK, ZCHUNK)],
                        rows[b].at[pl.ds(0, ZCHUNK)])
        pltpu.sync_copy(rows[b].at[pl.ds(0, ZCHUNK)],
                        accp_hbm.at[c, pl.ds(base + r * ZCHUNK, ZCHUNK)])


_segsum_kernel = pl.kernel(
    _segsum_body,
    out_type=jax.ShapeDtypeStruct((NC, N_NODES, D_CAT), jnp.float32),
    mesh=_mesh,
    scratch_types=[
        pltpu.VMEM((EPT,), jnp.int32),
        pltpu.VMEM((EPT,), jnp.int32),
        *[pltpu.VMEM((BB, D_CAT), jnp.float32) for _ in range(NBUF)],
        *[pltpu.SemaphoreType.DMA for _ in range(2 * NBUF)],
        pltpu.VMEM_SHARED((ACC_ROWS, D_CAT), jnp.float32),
        pltpu.VMEM_SHARED((N_NODES, D_CAT), jnp.float32),
    ],
    compiler_params=_sc_params,
)


def _post_body(accp_ref, g_ref, degp_ref, bmu_ref, bls_ref, mu_ref, ls_ref):
    dp = degp_ref[...]
    deg = dp[:, 0] + dp[:, 1] + 1.0
    dinv = lax.rsqrt(deg)
    ssum = accp_ref[0] + accp_ref[1] + g_ref[...]
    o = ssum * dinv[:, None]
    mu_ref[...] = o[:, :D_OUT] + bmu_ref[...]
    ls_ref[...] = o[:, D_OUT:] + bls_ref[...]


def _post(accp, g, degp, b_mu2, b_ls2):
    blk = 2000
    return pl.pallas_call(
        _post_body,
        grid=(N_NODES // blk,),
        in_specs=[
            pl.BlockSpec((NC, blk, D_CAT), lambda i: (0, i, 0)),
            pl.BlockSpec((blk, D_CAT), lambda i: (i, 0)),
            pl.BlockSpec((blk, NC), lambda i: (i, 0)),
            pl.BlockSpec((1, D_OUT), lambda i: (0, 0)),
            pl.BlockSpec((1, D_OUT), lambda i: (0, 0)),
        ],
        out_specs=[
            pl.BlockSpec((blk, D_OUT), lambda i: (i, 0)),
            pl.BlockSpec((blk, D_OUT), lambda i: (i, 0)),
        ],
        out_shape=[
            jax.ShapeDtypeStruct((N_NODES, D_OUT), jnp.float32),
            jax.ShapeDtypeStruct((N_NODES, D_OUT), jnp.float32),
        ],
    )(accp, g, degp, b_mu2, b_ls2)


@jax.jit
def kernel(x, edge_index, W_mu, b_mu, W_logstd, b_logstd):
    w_cat = jnp.concatenate([W_mu, W_logstd], axis=1)
    zeros1 = jnp.zeros((DEG_PER_TILE,), jnp.float32)
    ones1 = jnp.ones((BB,), jnp.float32)
    zeros2 = jnp.zeros((ZCHUNK, D_CAT), jnp.float32)

    degp = _deg_kernel(edge_index, zeros1, ones1)
    degp = degp[:, :N_NODES].T
    g = _pre(x, w_cat, degp)
    accp = _segsum_kernel(edge_index, g, zeros2)
    mu, ls = _post(accp, g, degp, b_mu.reshape(1, D_OUT), b_logstd.reshape(1, D_OUT))
    return mu, ls


# speedup vs baseline: 1.1029x; 1.1029x over previous
import jax
import jax.numpy as jnp
from jax import lax
from jax.experimental import pallas as pl
from jax.experimental.pallas import tpu as pltpu
from jax.experimental.pallas import tpu_sc as plsc

N_NODES = 10000
D_IN = 128
D_OUT = 32
D_CAT = 2 * D_OUT
N_EDGES = 320000

NC = 2
NS = 16
NW = NC * NS

EPT = N_EDGES // NW
BB = 128
NBF = EPT // BB
TAIL = EPT - NBF * BB
NBUF = 2

ACC_ROWS = N_NODES + 16
ROWS_PER_TILE = N_NODES // NS
ZCHUNK = 125

DEG_ROWS = 10240
DEG_PER_TILE = DEG_ROWS // NS

_mesh = plsc.VectorSubcoreMesh(
    core_axis_name="c", subcore_axis_name="s", num_cores=NC, num_subcores=NS
)

_sc_params = pltpu.CompilerParams(use_tc_tiling_on_sc=False)


NDSEM = 6


def _deg_body(edge_hbm, z_hbm, ones_hbm, degp_hbm, idx_v, zb_v, ones_v, *rest):
    dsem = rest[:NDSEM]
    deg_sp = rest[NDSEM]
    c = lax.axis_index("c")
    s = lax.axis_index("s")
    w = c * NS + s
    pltpu.sync_copy(edge_hbm.at[1, pl.ds(w * EPT, EPT)], idx_v)
    pltpu.sync_copy(z_hbm, zb_v)
    pltpu.sync_copy(ones_hbm, ones_v)
    pltpu.sync_copy(zb_v.at[pl.ds(0, DEG_PER_TILE)],
                    deg_sp.at[pl.ds(s * DEG_PER_TILE, DEG_PER_TILE)])
    plsc.subcore_barrier()

    def flight(u, carry):
        descs = []
        for b in range(NDSEM):
            j = u * NDSEM + b
            descs.append(pltpu.async_copy(
                ones_v, deg_sp.at[idx_v.at[pl.ds(j * BB, BB)]], dsem[b], add=True))
        for d in descs:
            d.wait()
        return carry

    lax.fori_loop(0, NBF // NDSEM, flight, 0)
    pltpu.sync_copy(ones_v.at[pl.ds(0, TAIL)],
                    deg_sp.at[idx_v.at[pl.ds(NBF * BB, TAIL)]], add=True)
    plsc.subcore_barrier()
    pltpu.sync_copy(deg_sp.at[pl.ds(s * DEG_PER_TILE, DEG_PER_TILE)], zb_v)
    pltpu.sync_copy(zb_v, degp_hbm.at[c, pl.ds(s * DEG_PER_TILE, DEG_PER_TILE)])


_deg_kernel = pl.kernel(
    _deg_body,
    out_type=jax.ShapeDtypeStruct((NC, DEG_ROWS), jnp.float32),
    mesh=_mesh,
    scratch_types=[
        pltpu.VMEM((EPT,), jnp.int32),
        pltpu.VMEM((DEG_PER_TILE,), jnp.float32),
        pltpu.VMEM((BB,), jnp.float32),
        *[pltpu.SemaphoreType.DMA for _ in range(NDSEM)],
        pltpu.VMEM_SHARED((DEG_ROWS,), jnp.float32),
    ],
    compiler_params=_sc_params,
)


def _pre_body(x_ref, w_ref, degp_ref, g_ref):
    dp = degp_ref[...]
    deg = dp[:, 0] + dp[:, 1] + 1.0
    dinv = lax.rsqrt(deg)
    h = jnp.dot(x_ref[...], w_ref[...], preferred_element_type=jnp.float32)
    g_ref[...] = h * dinv[:, None]


def _pre(x, w_cat, degp):
    blk = 2000
    return pl.pallas_call(
        _pre_body,
        grid=(N_NODES // blk,),
        in_specs=[
            pl.BlockSpec((blk, D_IN), lambda i: (i, 0)),
            pl.BlockSpec((D_IN, D_CAT), lambda i: (0, 0)),
            pl.BlockSpec((blk, NC), lambda i: (i, 0)),
        ],
        out_specs=pl.BlockSpec((blk, D_CAT), lambda i: (i, 0)),
        out_shape=jax.ShapeDtypeStruct((N_NODES, D_CAT), jnp.float32),
    )(x, w_cat, degp)


def _segsum_body(edge_hbm, g_hbm, z_hbm, accp_hbm, sidx_v, didx_v, *rest):
    rows = rest[:NBUF]
    gsem = rest[NBUF:2 * NBUF]
    ssem = rest[2 * NBUF:3 * NBUF]
    acc_sp = rest[3 * NBUF]
    g_sp = rest[3 * NBUF + 1]
    c = lax.axis_index("c")
    s = lax.axis_index("s")
    w = c * NS + s
    pltpu.sync_copy(edge_hbm.at[0, pl.ds(w * EPT, EPT)], sidx_v)
    pltpu.sync_copy(edge_hbm.at[1, pl.ds(w * EPT, EPT)], didx_v)
    base = s * ROWS_PER_TILE
    pltpu.sync_copy(g_hbm.at[pl.ds(base, ROWS_PER_TILE)],
                    g_sp.at[pl.ds(base, ROWS_PER_TILE)])
    pltpu.sync_copy(z_hbm, rows[0].at[pl.ds(0, ZCHUNK)])
    for r in range(ROWS_PER_TILE // ZCHUNK):
        pltpu.sync_copy(rows[0].at[pl.ds(0, ZCHUNK)],
                        acc_sp.at[pl.ds(base + r * ZCHUNK, ZCHUNK)])
    plsc.subcore_barrier()

    def gsrc(b):
        return g_sp if b % 2 == 0 else g_hbm

    for b in range(NBUF):
        pltpu.async_copy(gsrc(b).at[sidx_v.at[pl.ds(b * BB, BB)]], rows[b], gsem[b])

    def group(u, carry):
        sdescs = []
        for b in range(NBUF):
            j = u * NBUF + b
            pltpu.make_async_copy(
                gsrc(b).at[sidx_v.at[pl.ds(j * BB, BB)]], rows[b], gsem[b]).wait()
            sdescs.append(pltpu.async_copy(
                rows[b], acc_sp.at[didx_v.at[pl.ds(j * BB, BB)]], ssem[b], add=True))
        for b in range(NBUF):
            j2 = (u + 1) * NBUF + b
            sdescs[b].wait()
            pltpu.async_copy(gsrc(b).at[sidx_v.at[pl.ds(j2 * BB, BB)]], rows[b], gsem[b])
        return carry

    lax.fori_loop(0, NBF // NBUF - 1, group, 0)
    tdescs = []
    for b in range(NBUF):
        j = NBF - NBUF + b
        pltpu.make_async_copy(
            gsrc(b).at[sidx_v.at[pl.ds(j * BB, BB)]], rows[b], gsem[b]).wait()
        tdescs.append(pltpu.async_copy(
            rows[b], acc_sp.at[didx_v.at[pl.ds(j * BB, BB)]], ssem[b], add=True))
    for d in tdescs:
        d.wait()
    pltpu.sync_copy(g_sp.at[sidx_v.at[pl.ds(NBF * BB, TAIL)]],
                    rows[0].at[pl.ds(0, TAIL)])
    pltpu.sync_copy(rows[0].at[pl.ds(0, TAIL)],
                    acc_sp.at[didx_v.at[pl.ds(NBF * BB, TAIL)]], add=True)
    plsc.subcore_barrier()
    for r in range(ROWS_PER_TILE // ZCHUNK):
        b = r % NBUF
        pltpu.sync_copy(acc_sp.at[pl.ds(base + r * ZCHUNK, ZCHUNK)],
                        rows[b].at[pl.ds(0, ZCHUNK)])
        pltpu.sync_copy(rows[b].at[pl.ds(0, ZCHUNK)],
                        accp_hbm.at[c, pl.ds(base + r * ZCHUNK, ZCHUNK)])


_segsum_kernel = pl.kernel(
    _segsum_body,
    out_type=jax.ShapeDtypeStruct((NC, N_NODES, D_CAT), jnp.float32),
    mesh=_mesh,
    scratch_types=[
        pltpu.VMEM((EPT,), jnp.int32),
        pltpu.VMEM((EPT,), jnp.int32),
        *[pltpu.VMEM((BB, D_CAT), jnp.float32) for _ in range(NBUF)],
        *[pltpu.SemaphoreType.DMA for _ in range(2 * NBUF)],
        pltpu.VMEM_SHARED((ACC_ROWS, D_CAT), jnp.float32),
        pltpu.VMEM_SHARED((N_NODES, D_CAT), jnp.float32),
    ],
    compiler_params=_sc_params,
)


def _post_body(accp_ref, g_ref, degp_ref, bmu_ref, bls_ref, mu_ref, ls_ref):
    dp = degp_ref[...]
    deg = dp[:, 0] + dp[:, 1] + 1.0
    dinv = lax.rsqrt(deg)
    ssum = accp_ref[0] + accp_ref[1] + g_ref[...]
    o = ssum * dinv[:, None]
    mu_ref[...] = o[:, :D_OUT] + bmu_ref[...]
    ls_ref[...] = o[:, D_OUT:] + bls_ref[...]


def _post(accp, g, degp, b_mu2, b_ls2):
    blk = 2000
    return pl.pallas_call(
        _post_body,
        grid=(N_NODES // blk,),
        in_specs=[
            pl.BlockSpec((NC, blk, D_CAT), lambda i: (0, i, 0)),
            pl.BlockSpec((blk, D_CAT), lambda i: (i, 0)),
            pl.BlockSpec((blk, NC), lambda i: (i, 0)),
            pl.BlockSpec((1, D_OUT), lambda i: (0, 0)),
            pl.BlockSpec((1, D_OUT), lambda i: (0, 0)),
        ],
        out_specs=[
            pl.BlockSpec((blk, D_OUT), lambda i: (i, 0)),
            pl.BlockSpec((blk, D_OUT), lambda i: (i, 0)),
        ],
        out_shape=[
            jax.ShapeDtypeStruct((N_NODES, D_OUT), jnp.float32),
            jax.ShapeDtypeStruct((N_NODES, D_OUT), jnp.float32),
        ],
    )(accp, g, degp, b_mu2, b_ls2)


@jax.jit
def kernel(x, edge_index, W_mu, b_mu, W_logstd, b_logstd):
    w_cat = jnp.concatenate([W_mu, W_logstd], axis=1)
    zeros1 = jnp.zeros((DEG_PER_TILE,), jnp.float32)
    ones1 = jnp.ones((BB,), jnp.float32)
    zeros2 = jnp.zeros((ZCHUNK, D_CAT), jnp.float32)

    degp = _deg_kernel(edge_index, zeros1, ones1)
    degp = degp[:, :N_NODES].T
    g = _pre(x, w_cat, degp)
    accp = _segsum_kernel(edge_index, g, zeros2)
    mu, ls = _post(accp, g, degp, b_mu.reshape(1, D_OUT), b_logstd.reshape(1, D_OUT))
    return mu, ls
